# baseline (speedup 1.0000x reference)
import functools

import jax
import jax.numpy as jnp
from jax import lax
from jax.experimental import pallas as pl
from jax.experimental.pallas import tpu as pltpu
from jax.experimental.pallas import tpu_sc as plsc

_N = 10000
_E = 320000
_D = 128
_DE = 16
_NG = 64
_NCLS = 10

_NCORE = 2
_NSUB = 16
_CH = 128
_SB = 8
_MC0 = 120
_MC1 = 40
_NCHUNK = 80
_NCHTOT = _NCORE * _NSUB * _NCHUNK
_NCHREAL = _E // _CH
_NPAD = 10240
_STRIPE = _NPAD // _NSUB

_R = 1000
_GRID = _N // _R


def _node_pro_body(x_ref, w_ref, av_ref, ad_ref, xp_ref, xpb_ref, asr_ref, adt_ref):
    xp = jnp.dot(x_ref[...], w_ref[...], preferred_element_type=jnp.float32)
    xp_ref[...] = xp
    xpb_ref[...] = xp.astype(jnp.bfloat16)
    asr_ref[...] = jnp.dot(xp, av_ref[...], preferred_element_type=jnp.float32)
    adt_ref[...] = jnp.dot(xp, ad_ref[...], preferred_element_type=jnp.float32)


def _node_pro(x, W, av, ad):
    return pl.pallas_call(
        _node_pro_body,
        grid=(_GRID,),
        in_specs=[
            pl.BlockSpec((_R, _D), lambda i: (i, 0)),
            pl.BlockSpec((_D, _D), lambda i: (0, 0)),
            pl.BlockSpec((_D, 1), lambda i: (0, 0)),
            pl.BlockSpec((_D, 1), lambda i: (0, 0)),
        ],
        out_specs=[
            pl.BlockSpec((_R, _D), lambda i: (i, 0)),
            pl.BlockSpec((_R, _D), lambda i: (i, 0)),
            pl.BlockSpec((_R, 1), lambda i: (i, 0)),
            pl.BlockSpec((_R, 1), lambda i: (i, 0)),
        ],
        out_shape=[
            jax.ShapeDtypeStruct((_N, _D), jnp.float32),
            jax.ShapeDtypeStruct((_N, _D), jnp.bfloat16),
            jax.ShapeDtypeStruct((_N, 1), jnp.float32),
            jax.ShapeDtypeStruct((_N, 1), jnp.float32),
        ],
    )(x, W, av, ad)


_EB = 8192


def _edge_pro_body(ea_ref, we1_ref, ae1_ref, we2_ref, ae2_ref, o1_ref, o2_ref):
    i = pl.program_id(0)
    eidx = i * _EB + lax.broadcasted_iota(jnp.int32, (_EB, 1), 0)
    real = eidx < _E
    ea = ea_ref[...]
    for we_ref, atte_ref, o_ref in ((we1_ref, ae1_ref, o1_ref),
                                    (we2_ref, ae2_ref, o2_ref)):
        wv = jnp.dot(we_ref[...], atte_ref[...], preferred_element_type=jnp.float32)
        ae = jnp.dot(ea, wv, preferred_element_type=jnp.float32)
        o_ref[...] = jnp.where(real, ae, -1e30)


def _edge_pro(edge_attr, We1, atte1, We2, atte2):
    return pl.pallas_call(
        _edge_pro_body,
        grid=(_NCHTOT * _CH // _EB,),
        in_specs=[
            pl.BlockSpec((_EB, _DE), lambda i: (i, 0)),
            pl.BlockSpec((_DE, _D), lambda i: (0, 0)),
            pl.BlockSpec((_D, 1), lambda i: (0, 0)),
            pl.BlockSpec((_DE, _D), lambda i: (0, 0)),
            pl.BlockSpec((_D, 1), lambda i: (0, 0)),
        ],
        out_specs=[pl.BlockSpec((_EB, 1), lambda i: (i, 0))] * 2,
        out_shape=[jax.ShapeDtypeStruct((_NCHTOT * _CH, 1), jnp.float32)] * 2,
    )(edge_attr, We1, atte1, We2, atte2)


def _layout_body(src_ref, dst_ref, srco_ref, dsto_ref):
    i = pl.program_id(0)
    rid = i * 512 + lax.broadcasted_iota(jnp.int32, (512, _CH), 0)
    real = rid < _NCHREAL
    srco_ref[...] = jnp.where(real, src_ref[...], 0)
    dsto_ref[...] = jnp.where(real, dst_ref[...], _N)


def _layout(src2d, dst2d):
    blk = pl.BlockSpec((512, _CH), lambda i: (i, 0))
    return pl.pallas_call(
        _layout_body,
        grid=(_NCHTOT // 512,),
        in_specs=[blk] * 2,
        out_specs=[blk] * 2,
        out_shape=[
            jax.ShapeDtypeStruct((_NCHTOT, _CH), jnp.int32),
            jax.ShapeDtypeStruct((_NCHTOT, _CH), jnp.int32),
        ],
    )(src2d, dst2d)


def _epilogue(acc0, acc1, den0, den1, sae0, sae1, dg0, dg1, asr, adt, xp, b_ref):
    def m2(r):
        return r[...].reshape(_R, _D).astype(jnp.float32)

    def v2(r):
        return r[...].reshape(_R, 1)

    deg = jnp.maximum(v2(dg0) + v2(dg1), 1.0)
    aloop = asr[...] + adt[...] + (v2(sae0) + v2(sae1)) / deg
    aloop = jnp.where(aloop >= 0.0, aloop, 0.2 * aloop)
    ploop = jnp.exp(aloop)
    invd = 1.0 / (v2(den0) + v2(den1) + ploop + 1e-16)
    h = (m2(acc0) + m2(acc1) + ploop * xp[...]) * invd + b_ref[...]
    return jnp.maximum(h, 0.0)


def _tc_mid_body(acc0, acc1, den0, den1, sae0, sae1, dg0, dg1, asr, adt, xp,
                 b_ref, w2_ref, av2_ref, ad2_ref, xp2_ref, xp2b_ref, as2_ref,
                 ad2o_ref):
    h = _epilogue(acc0, acc1, den0, den1, sae0, sae1, dg0, dg1, asr, adt, xp, b_ref)
    xp2 = jnp.dot(h, w2_ref[...], preferred_element_type=jnp.float32)
    xp2_ref[...] = xp2
    xp2b_ref[...] = xp2.astype(jnp.bfloat16)
    as2_ref[...] = jnp.dot(xp2, av2_ref[...], preferred_element_type=jnp.float32)
    ad2o_ref[...] = jnp.dot(xp2, ad2_ref[...], preferred_element_type=jnp.float32)


def _core_specs():
    acc0 = pl.BlockSpec((1, _R, _D), lambda i: (0, i, 0))
    acc1 = pl.BlockSpec((1, _R, _D), lambda i: (1, i, 0))
    v0 = pl.BlockSpec((1, _R, 1), lambda i: (0, i, 0))
    v1 = pl.BlockSpec((1, _R, 1), lambda i: (1, i, 0))
    return [acc0, acc1, v0, v1, v0, v1, v0, v1]


def _tc_mid(acc, den, sae, dg, asr, adt, xp, b, W2, av2, ad2):
    vec = pl.BlockSpec((_R, 1), lambda i: (i, 0))
    mat = pl.BlockSpec((_R, _D), lambda i: (i, 0))
    return pl.pallas_call(
        _tc_mid_body,
        grid=(_GRID,),
        in_specs=_core_specs() + [vec, vec] + [
            mat,
            pl.BlockSpec((1, _D), lambda i: (0, 0)),
            pl.BlockSpec((_D, _D), lambda i: (0, 0)),
            pl.BlockSpec((_D, 1), lambda i: (0, 0)),
            pl.BlockSpec((_D, 1), lambda i: (0, 0)),
        ],
        out_specs=[mat, mat, vec, vec],
        out_shape=[
            jax.ShapeDtypeStruct((_N, _D), jnp.float32),
            jax.ShapeDtypeStruct((_N, _D), jnp.bfloat16),
            jax.ShapeDtypeStruct((_N, 1), jnp.float32),
            jax.ShapeDtypeStruct((_N, 1), jnp.float32),
        ],
    )(acc, acc, den, den, sae, sae, dg, dg, asr, adt, xp, b, W2, av2, ad2)


def _tc_fin_body(acc0, acc1, den0, den1, sae0, sae1, dg0, dg1, asr, adt, xp,
                 b_ref, batch_ref, wf_ref, bf_ref, out_ref, pooled_s, cnt_s):
    i = pl.program_id(0)
    h = _epilogue(acc0, acc1, den0, den1, sae0, sae1, dg0, dg1, asr, adt, xp, b_ref)
    oh = (batch_ref[...] == lax.broadcasted_iota(jnp.int32, (_R, _NG), 1)
          ).astype(jnp.float32)

    @pl.when(i == 0)
    def _():
        pooled_s[...] = jnp.zeros_like(pooled_s)
        cnt_s[...] = jnp.zeros_like(cnt_s)

    dn = (((0,), (0,)), ((), ()))
    pooled_s[...] += lax.dot_general(oh, h, dn, preferred_element_type=jnp.float32)
    cnt_s[...] += lax.dot_general(oh, jnp.ones((_R, _D), jnp.float32), dn,
                                  preferred_element_type=jnp.float32)

    @pl.when(i == _GRID - 1)
    def _():
        pooled = pooled_s[...] / jnp.maximum(cnt_s[...], 1.0)
        logits = jnp.dot(pooled, wf_ref[...], preferred_element_type=jnp.float32) + bf_ref[...]
        m = jnp.max(logits, axis=1, keepdims=True)
        e = jnp.exp(logits - m)
        out_ref[...] = e / jnp.sum(e, axis=1, keepdims=True)


def _tc_fin(acc, den, sae, dg, asr, adt, xp, b, batch, Wf, bf):
    vec = pl.BlockSpec((_R, 1), lambda i: (i, 0))
    mat = pl.BlockSpec((_R, _D), lambda i: (i, 0))
    return pl.pallas_call(
        _tc_fin_body,
        grid=(_GRID,),
        in_specs=_core_specs() + [vec, vec] + [
            mat,
            pl.BlockSpec((1, _D), lambda i: (0, 0)),
            pl.BlockSpec((_R, 1), lambda i: (i, 0)),
            pl.BlockSpec((_D, _NCLS), lambda i: (0, 0)),
            pl.BlockSpec((1, _NCLS), lambda i: (0, 0)),
        ],
        out_specs=pl.BlockSpec((_NG, _NCLS), lambda i: (0, 0)),
        out_shape=jax.ShapeDtypeStruct((_NG, _NCLS), jnp.float32),
        scratch_shapes=[
            pltpu.VMEM((_NG, _D), jnp.float32),
            pltpu.VMEM((_NG, _D), jnp.float32),
        ],
    )(acc, acc, den, den, sae, sae, dg, dg, asr, adt, xp, b, batch, Wf, bf)


def _sc_scores_body(src_hbm, dst_hbm, ae_hbm, asrc_hbm, adst_hbm,
                    p_o, den_o, sae_o, deg_o,
                    src_v, dst_v, ae_v, p_v, ones_v, asrc_v, adst_v, zvec_v,
                    den_s, sae_s, deg_s, ssem):
    c = lax.axis_index("c")
    s = lax.axis_index("s")
    base = s * _STRIPE
    cb = (c * _NSUB + s) * _NCHUNK

    pltpu.sync_copy(src_hbm.at[pl.ds(cb, _NCHUNK)], src_v)
    pltpu.sync_copy(dst_hbm.at[pl.ds(cb, _NCHUNK)], dst_v)
    pltpu.sync_copy(ae_hbm.at[pl.ds(cb * _CH, _NCHUNK * _CH)], ae_v)
    pltpu.sync_copy(asrc_hbm, asrc_v)
    pltpu.sync_copy(adst_hbm, adst_v)

    def _zvec(i, carry):
        zvec_v[pl.ds(i * 16, 16)] = jnp.zeros((16,), jnp.float32)
        return carry
    lax.fori_loop(0, _STRIPE // 16, _zvec, 0)
    for g in range(8):
        ones_v[pl.ds(g * 16, 16)] = jnp.ones((16,), jnp.float32)

    pltpu.sync_copy(zvec_v, den_s.at[pl.ds(base, _STRIPE)])
    pltpu.sync_copy(zvec_v, sae_s.at[pl.ds(base, _STRIPE)])
    pltpu.sync_copy(zvec_v, deg_s.at[pl.ds(base, _STRIPE)])
    plsc.subcore_barrier()

    def _chunk(j, carry):
        for g in range(8):
            sl = pl.ds(g * 16, 16)
            si = src_v[j, sl]
            di = dst_v[j, sl]
            r = (plsc.load_gather(asrc_v, [si]) + plsc.load_gather(adst_v, [di])
                 + ae_v[pl.ds(j * _CH + g * 16, 16)])
            r = jnp.where(r >= 0.0, r, 0.2 * r)
            p_v[j, sl] = jnp.exp(r)
        pltpu.async_copy(p_v.at[j], den_s.at[dst_v.at[j]], ssem, add=True)
        pltpu.async_copy(ae_v.at[pl.ds(j * _CH, _CH)], sae_s.at[dst_v.at[j]],
                         ssem, add=True)
        pltpu.async_copy(ones_v, deg_s.at[dst_v.at[j]], ssem, add=True)
        return carry
    lax.fori_loop(0, _NCHUNK, _chunk, 0)

    pltpu.sync_copy(p_v, p_o.at[pl.ds(cb, _NCHUNK)])

    def _drain(j, carry):
        pltpu.make_async_copy(p_v.at[j], den_s.at[dst_v.at[j]], ssem).wait()
        pltpu.make_async_copy(ae_v.at[pl.ds(j * _CH, _CH)],
                              sae_s.at[dst_v.at[j]], ssem).wait()
        pltpu.make_async_copy(ones_v, deg_s.at[dst_v.at[j]], ssem).wait()
        return carry
    lax.fori_loop(0, _NCHUNK, _drain, 0)

    plsc.subcore_barrier()
    pltpu.sync_copy(den_s.at[pl.ds(base, _STRIPE)],
                    den_o.at[c, pl.ds(base, _STRIPE)])
    pltpu.sync_copy(sae_s.at[pl.ds(base, _STRIPE)],
                    sae_o.at[c, pl.ds(base, _STRIPE)])
    pltpu.sync_copy(deg_s.at[pl.ds(base, _STRIPE)],
                    deg_o.at[c, pl.ds(base, _STRIPE)])


_sc_scores = pl.kernel(
    _sc_scores_body,
    out_type=(
        jax.ShapeDtypeStruct((_NCHTOT, _CH), jnp.float32),
        jax.ShapeDtypeStruct((_NCORE, _NPAD), jnp.float32),
        jax.ShapeDtypeStruct((_NCORE, _NPAD), jnp.float32),
        jax.ShapeDtypeStruct((_NCORE, _NPAD), jnp.float32),
    ),
    mesh=plsc.VectorSubcoreMesh(core_axis_name="c", subcore_axis_name="s"),
    scratch_types=[
        pltpu.VMEM((_NCHUNK, _CH), jnp.int32),
        pltpu.VMEM((_NCHUNK, _CH), jnp.int32),
        pltpu.VMEM((_NCHUNK * _CH,), jnp.float32),
        pltpu.VMEM((_NCHUNK, _CH), jnp.float32),
        pltpu.VMEM((_CH,), jnp.float32),
        pltpu.VMEM((_NPAD,), jnp.float32),
        pltpu.VMEM((_NPAD,), jnp.float32),
        pltpu.VMEM((_STRIPE,), jnp.float32),
        pltpu.VMEM_SHARED((_NPAD,), jnp.float32),
        pltpu.VMEM_SHARED((_NPAD,), jnp.float32),
        pltpu.VMEM_SHARED((_NPAD,), jnp.float32),
        pltpu.SemaphoreType.DMA,
    ],
    compiler_params=pltpu.CompilerParams(needs_layout_passes=False),
)


def _sc_msgs_body(src_hbm, dst_hbm, p_hbm, xp_hbm, acc_o,
                  src_v, dst_v, p_v, rows0_v, rows1_v, acc_s, gsem0, gsem1):
    c = lax.axis_index("c")
    s = lax.axis_index("s")
    base = s * _STRIPE
    rows = (rows0_v, rows1_v)
    gsem = (gsem0, gsem1)

    def _zrow(i, carry):
        for g in range(4):
            rows0_v[i, pl.ds(g * 32, 32)] = jnp.zeros((32,), jnp.bfloat16)
        return carry
    lax.fori_loop(0, _CH, _zrow, 0)
    for k in range(_STRIPE // _CH):
        pltpu.sync_copy(rows0_v, acc_s.at[pl.ds(base + k * _CH, _CH)])
    plsc.subcore_barrier()

    def _run(cb0, nsb):
        def _sblock(sb, carry):
            sb0 = cb0 + sb * _SB
            pltpu.sync_copy(src_hbm.at[pl.ds(sb0, _SB)], src_v)
            pltpu.sync_copy(dst_hbm.at[pl.ds(sb0, _SB)], dst_v)
            pltpu.sync_copy(p_hbm.at[pl.ds(sb0, _SB)], p_v)
            pltpu.async_copy(xp_hbm.at[src_v.at[0]], rows0_v, gsem0)

            def _pair(pr, carry2):
                for b in range(2):
                    j = pr * 2 + b
                    rb = rows[b]
                    pltpu.make_async_copy(xp_hbm.at[src_v.at[j]], rb,
                                          gsem[b]).wait()

                    @pl.when(j < _SB - 1)
                    def _():
                        pltpu.async_copy(xp_hbm.at[src_v.at[j + 1]],
                                         rows[1 - b], gsem[1 - b])

                    @plsc.parallel_loop(0, _CH, unroll=4)
                    def _rowscale(rr):
                        jj = jnp.full((16,), j, jnp.int32)
                        pb = plsc.load_gather(
                            p_v, [jj, jnp.full((16,), rr, jnp.int32)])
                        pbb = plsc.pack(pb, pb, format=plsc.PackFormat.INTERLEAVED)
                        for g in range(4):
                            sl = pl.ds(g * 32, 32)
                            rb[rr, sl] = rb[rr, sl] * pbb

                    pltpu.sync_copy(rb, acc_s.at[dst_v.at[j]], add=True)
                return carry2
            lax.fori_loop(0, _SB // 2, _pair, 0)
            return carry
        lax.fori_loop(0, nsb, _sblock, 0)

    @pl.when(c == 0)
    def _():
        _run(s * _MC0, _MC0 // _SB)

    @pl.when(c == 1)
    def _():
        _run(_NSUB * _MC0 + s * _MC1, _MC1 // _SB)

    plsc.subcore_barrier()
    pltpu.sync_copy(acc_s.at[pl.ds(base, _STRIPE)], acc_o.at[c, pl.ds(base, _STRIPE)])


_sc_msgs = pl.kernel(
    _sc_msgs_body,
    out_type=jax.ShapeDtypeStruct((_NCORE, _NPAD, _D), jnp.bfloat16),
    mesh=plsc.VectorSubcoreMesh(core_axis_name="c", subcore_axis_name="s"),
    scratch_types=[
        pltpu.VMEM((_SB, _CH), jnp.int32),
        pltpu.VMEM((_SB, _CH), jnp.int32),
        pltpu.VMEM((_SB, _CH), jnp.float32),
        pltpu.VMEM((_CH, _D), jnp.bfloat16),
        pltpu.VMEM((_CH, _D), jnp.bfloat16),
        pltpu.VMEM_SHARED((_NPAD, _D), jnp.bfloat16),
        pltpu.SemaphoreType.DMA,
        pltpu.SemaphoreType.DMA,
    ],
    compiler_params=pltpu.CompilerParams(needs_layout_passes=False,
                                         use_tc_tiling_on_sc=False),
)


def kernel(x, edge_index, edge_attr, batch, W1, att_src1, att_dst1, We1, att_e1,
           b1, W2, att_src2, att_dst2, We2, att_e2, b2, Wf, bf):
    ae1, ae2 = _edge_pro(edge_attr, We1, att_e1.reshape(_D, 1), We2,
                         att_e2.reshape(_D, 1))
    ae1 = ae1.reshape(_NCHTOT * _CH)
    ae2 = ae2.reshape(_NCHTOT * _CH)
    src, dst = _layout(
        edge_index[0].astype(jnp.int32).reshape(_NCHREAL, _CH),
        edge_index[1].astype(jnp.int32).reshape(_NCHREAL, _CH))

    def pad_n(v):
        return jnp.concatenate([v.reshape(_N), jnp.zeros((_NPAD - _N,), jnp.float32)])

    def v3(a):
        return a.reshape(_NCORE, _NPAD, 1)

    xp1, xp1b, asr1, adt1 = _node_pro(x, W1, att_src1.reshape(_D, 1),
                                      att_dst1.reshape(_D, 1))
    p1, den1, sae1, deg1 = _sc_scores(src, dst, ae1, pad_n(asr1), pad_n(adt1))
    acc1 = _sc_msgs(src, dst, p1, xp1b)

    xp2, xp2b, asr2, adt2 = _tc_mid(
        acc1, v3(den1), v3(sae1), v3(deg1), asr1, adt1, xp1,
        b1.reshape(1, _D), W2, att_src2.reshape(_D, 1), att_dst2.reshape(_D, 1))

    p2, den2, sae2, _ = _sc_scores(src, dst, ae2, pad_n(asr2), pad_n(adt2))
    acc2 = _sc_msgs(src, dst, p2, xp2b)

    return _tc_fin(
        acc2, v3(den2), v3(sae2), v3(deg1), asr2, adt2, xp2,
        b2.reshape(1, _D), batch.astype(jnp.int32).reshape(_N, 1), Wf,
        bf.reshape(1, _NCLS))


# speedup vs baseline: 1.1466x; 1.1466x over previous
import functools

import jax
import jax.numpy as jnp
from jax import lax
from jax.experimental import pallas as pl
from jax.experimental.pallas import tpu as pltpu
from jax.experimental.pallas import tpu_sc as plsc

_N = 10000
_E = 320000
_D = 128
_DE = 16
_NG = 64
_NCLS = 10

_NCORE = 2
_NSUB = 16
_CH = 128
_SB = 8
_MC0 = 120
_MC1 = 40
_NCHUNK = 80
_NCHTOT = _NCORE * _NSUB * _NCHUNK
_NCHREAL = _E // _CH
_NPAD = 10240
_STRIPE = _NPAD // _NSUB

_R = 1000
_GRID = _N // _R


def _node_pro_body(x_ref, w_ref, av_ref, ad_ref, xp_ref, xpb_ref, asr_ref, adt_ref):
    xp = jnp.dot(x_ref[...], w_ref[...], preferred_element_type=jnp.float32)
    xp_ref[...] = xp
    xpb_ref[...] = xp.astype(jnp.bfloat16)
    asr_ref[...] = jnp.dot(xp, av_ref[...], preferred_element_type=jnp.float32)
    adt_ref[...] = jnp.dot(xp, ad_ref[...], preferred_element_type=jnp.float32)


def _node_pro(x, W, av, ad):
    return pl.pallas_call(
        _node_pro_body,
        grid=(_GRID,),
        in_specs=[
            pl.BlockSpec((_R, _D), lambda i: (i, 0)),
            pl.BlockSpec((_D, _D), lambda i: (0, 0)),
            pl.BlockSpec((_D, 1), lambda i: (0, 0)),
            pl.BlockSpec((_D, 1), lambda i: (0, 0)),
        ],
        out_specs=[
            pl.BlockSpec((_R, _D), lambda i: (i, 0)),
            pl.BlockSpec((_R, _D), lambda i: (i, 0)),
            pl.BlockSpec((_R, 1), lambda i: (i, 0)),
            pl.BlockSpec((_R, 1), lambda i: (i, 0)),
        ],
        out_shape=[
            jax.ShapeDtypeStruct((_N, _D), jnp.float32),
            jax.ShapeDtypeStruct((_N, _D), jnp.bfloat16),
            jax.ShapeDtypeStruct((_N, 1), jnp.float32),
            jax.ShapeDtypeStruct((_N, 1), jnp.float32),
        ],
    )(x, W, av, ad)


_EB = 8192


def _edge_pro_body(ea_ref, we1_ref, ae1_ref, we2_ref, ae2_ref, o1_ref, o2_ref):
    i = pl.program_id(0)
    eidx = i * _EB + lax.broadcasted_iota(jnp.int32, (_EB, 1), 0)
    real = eidx < _E
    er = lax.broadcasted_iota(jnp.int32, (_EB, _CH), 0)
    bsel = (er % _CH) == lax.broadcasted_iota(jnp.int32, (_EB, _CH), 1)
    amat = ((lax.broadcasted_iota(jnp.int32, (_EB // _CH, _EB), 1) // _CH)
            == lax.broadcasted_iota(jnp.int32, (_EB // _CH, _EB), 0)
            ).astype(jnp.float32)
    ea = ea_ref[...]
    for we_ref, atte_ref, o_ref in ((we1_ref, ae1_ref, o1_ref),
                                    (we2_ref, ae2_ref, o2_ref)):
        wv = jnp.dot(we_ref[...], atte_ref[...], preferred_element_type=jnp.float32)
        v = jnp.where(real, jnp.dot(ea, wv, preferred_element_type=jnp.float32),
                      -1e30)
        vb = jnp.where(bsel, v, 0.0)
        o_ref[...] = jnp.dot(amat, vb, preferred_element_type=jnp.float32)


def _edge_pro(edge_attr, We1, atte1, We2, atte2):
    return pl.pallas_call(
        _edge_pro_body,
        grid=(_NCHTOT * _CH // _EB,),
        in_specs=[
            pl.BlockSpec((_EB, _DE), lambda i: (i, 0)),
            pl.BlockSpec((_DE, _D), lambda i: (0, 0)),
            pl.BlockSpec((_D, 1), lambda i: (0, 0)),
            pl.BlockSpec((_DE, _D), lambda i: (0, 0)),
            pl.BlockSpec((_D, 1), lambda i: (0, 0)),
        ],
        out_specs=[pl.BlockSpec((_EB // _CH, _CH), lambda i: (i, 0))] * 2,
        out_shape=[jax.ShapeDtypeStruct((_NCHTOT, _CH), jnp.float32)] * 2,
    )(edge_attr, We1, atte1, We2, atte2)


def _layout_body(src_ref, dst_ref, srco_ref, dsto_ref):
    i = pl.program_id(0)
    rid = i * 512 + lax.broadcasted_iota(jnp.int32, (512, _CH), 0)
    real = rid < _NCHREAL
    srco_ref[...] = jnp.where(real, src_ref[...], 0)
    dsto_ref[...] = jnp.where(real, dst_ref[...], _N)


def _layout(src2d, dst2d):
    blk = pl.BlockSpec((512, _CH), lambda i: (i, 0))
    return pl.pallas_call(
        _layout_body,
        grid=(_NCHTOT // 512,),
        in_specs=[blk] * 2,
        out_specs=[blk] * 2,
        out_shape=[
            jax.ShapeDtypeStruct((_NCHTOT, _CH), jnp.int32),
            jax.ShapeDtypeStruct((_NCHTOT, _CH), jnp.int32),
        ],
    )(src2d, dst2d)


def _epilogue(acc0, acc1, den0, den1, sae0, sae1, dg0, dg1, asr, adt, xp, b_ref):
    def m2(r):
        return r[...].reshape(_R, _D).astype(jnp.float32)

    def v2(r):
        return r[...].reshape(_R, 1)

    deg = jnp.maximum(v2(dg0) + v2(dg1), 1.0)
    aloop = asr[...] + adt[...] + (v2(sae0) + v2(sae1)) / deg
    aloop = jnp.where(aloop >= 0.0, aloop, 0.2 * aloop)
    ploop = jnp.exp(aloop)
    invd = 1.0 / (v2(den0) + v2(den1) + ploop + 1e-16)
    h = (m2(acc0) + m2(acc1) + ploop * xp[...]) * invd + b_ref[...]
    return jnp.maximum(h, 0.0)


def _tc_mid_body(acc0, acc1, den0, den1, sae0, sae1, dg0, dg1, asr, adt, xp,
                 b_ref, w2_ref, av2_ref, ad2_ref, xp2_ref, xp2b_ref, as2_ref,
                 ad2o_ref):
    h = _epilogue(acc0, acc1, den0, den1, sae0, sae1, dg0, dg1, asr, adt, xp, b_ref)
    xp2 = jnp.dot(h, w2_ref[...], preferred_element_type=jnp.float32)
    xp2_ref[...] = xp2
    xp2b_ref[...] = xp2.astype(jnp.bfloat16)
    as2_ref[...] = jnp.dot(xp2, av2_ref[...], preferred_element_type=jnp.float32)
    ad2o_ref[...] = jnp.dot(xp2, ad2_ref[...], preferred_element_type=jnp.float32)


def _core_specs():
    acc0 = pl.BlockSpec((1, _R, _D), lambda i: (0, i, 0))
    acc1 = pl.BlockSpec((1, _R, _D), lambda i: (1, i, 0))
    v0 = pl.BlockSpec((1, _R, 1), lambda i: (0, i, 0))
    v1 = pl.BlockSpec((1, _R, 1), lambda i: (1, i, 0))
    return [acc0, acc1, v0, v1, v0, v1, v0, v1]


def _tc_mid(acc, den, sae, dg, asr, adt, xp, b, W2, av2, ad2):
    vec = pl.BlockSpec((_R, 1), lambda i: (i, 0))
    mat = pl.BlockSpec((_R, _D), lambda i: (i, 0))
    return pl.pallas_call(
        _tc_mid_body,
        grid=(_GRID,),
        in_specs=_core_specs() + [vec, vec] + [
            mat,
            pl.BlockSpec((1, _D), lambda i: (0, 0)),
            pl.BlockSpec((_D, _D), lambda i: (0, 0)),
            pl.BlockSpec((_D, 1), lambda i: (0, 0)),
            pl.BlockSpec((_D, 1), lambda i: (0, 0)),
        ],
        out_specs=[mat, mat, vec, vec],
        out_shape=[
            jax.ShapeDtypeStruct((_N, _D), jnp.float32),
            jax.ShapeDtypeStruct((_N, _D), jnp.bfloat16),
            jax.ShapeDtypeStruct((_N, 1), jnp.float32),
            jax.ShapeDtypeStruct((_N, 1), jnp.float32),
        ],
    )(acc, acc, den, den, sae, sae, dg, dg, asr, adt, xp, b, W2, av2, ad2)


def _tc_fin_body(acc0, acc1, den0, den1, sae0, sae1, dg0, dg1, asr, adt, xp,
                 b_ref, batch_ref, wf_ref, bf_ref, out_ref, pooled_s, cnt_s):
    i = pl.program_id(0)
    h = _epilogue(acc0, acc1, den0, den1, sae0, sae1, dg0, dg1, asr, adt, xp, b_ref)
    oh = (batch_ref[...] == lax.broadcasted_iota(jnp.int32, (_R, _NG), 1)
          ).astype(jnp.float32)

    @pl.when(i == 0)
    def _():
        pooled_s[...] = jnp.zeros_like(pooled_s)
        cnt_s[...] = jnp.zeros_like(cnt_s)

    dn = (((0,), (0,)), ((), ()))
    pooled_s[...] += lax.dot_general(oh, h, dn, preferred_element_type=jnp.float32)
    cnt_s[...] += lax.dot_general(oh, jnp.ones((_R, _D), jnp.float32), dn,
                                  preferred_element_type=jnp.float32)

    @pl.when(i == _GRID - 1)
    def _():
        pooled = pooled_s[...] / jnp.maximum(cnt_s[...], 1.0)
        logits = jnp.dot(pooled, wf_ref[...], preferred_element_type=jnp.float32) + bf_ref[...]
        m = jnp.max(logits, axis=1, keepdims=True)
        e = jnp.exp(logits - m)
        out_ref[...] = e / jnp.sum(e, axis=1, keepdims=True)


def _tc_fin(acc, den, sae, dg, asr, adt, xp, b, batch, Wf, bf):
    vec = pl.BlockSpec((_R, 1), lambda i: (i, 0))
    mat = pl.BlockSpec((_R, _D), lambda i: (i, 0))
    return pl.pallas_call(
        _tc_fin_body,
        grid=(_GRID,),
        in_specs=_core_specs() + [vec, vec] + [
            mat,
            pl.BlockSpec((1, _D), lambda i: (0, 0)),
            pl.BlockSpec((_R, 1), lambda i: (i, 0)),
            pl.BlockSpec((_D, _NCLS), lambda i: (0, 0)),
            pl.BlockSpec((1, _NCLS), lambda i: (0, 0)),
        ],
        out_specs=pl.BlockSpec((_NG, _NCLS), lambda i: (0, 0)),
        out_shape=jax.ShapeDtypeStruct((_NG, _NCLS), jnp.float32),
        scratch_shapes=[
            pltpu.VMEM((_NG, _D), jnp.float32),
            pltpu.VMEM((_NG, _D), jnp.float32),
        ],
    )(acc, acc, den, den, sae, sae, dg, dg, asr, adt, xp, b, batch, Wf, bf)


def _sc_scores_body(src_hbm, dst_hbm, ae_hbm, asrc_hbm, adst_hbm,
                    p_o, den_o, sae_o, deg_o,
                    src_v, dst_v, ae_v, p_v, ones_v, asrc_v, adst_v, zvec_v,
                    den_s, sae_s, deg_s, ssem):
    c = lax.axis_index("c")
    s = lax.axis_index("s")
    base = s * _STRIPE
    cb = (c * _NSUB + s) * _NCHUNK

    pltpu.sync_copy(src_hbm.at[pl.ds(cb, _NCHUNK)], src_v)
    pltpu.sync_copy(dst_hbm.at[pl.ds(cb, _NCHUNK)], dst_v)
    pltpu.sync_copy(ae_hbm.at[pl.ds(cb, _NCHUNK)], ae_v)
    pltpu.sync_copy(asrc_hbm, asrc_v)
    pltpu.sync_copy(adst_hbm, adst_v)

    def _zvec(i, carry):
        zvec_v[pl.ds(i * 16, 16)] = jnp.zeros((16,), jnp.float32)
        return carry
    lax.fori_loop(0, _STRIPE // 16, _zvec, 0)
    for g in range(8):
        ones_v[pl.ds(g * 16, 16)] = jnp.ones((16,), jnp.float32)

    pltpu.sync_copy(zvec_v, den_s.at[pl.ds(base, _STRIPE)])
    pltpu.sync_copy(zvec_v, sae_s.at[pl.ds(base, _STRIPE)])
    pltpu.sync_copy(zvec_v, deg_s.at[pl.ds(base, _STRIPE)])
    plsc.subcore_barrier()

    def _chunk(j, carry):
        for g in range(8):
            sl = pl.ds(g * 16, 16)
            si = src_v[j, sl]
            di = dst_v[j, sl]
            r = (plsc.load_gather(asrc_v, [si]) + plsc.load_gather(adst_v, [di])
                 + ae_v[j, sl])
            r = jnp.where(r >= 0.0, r, 0.2 * r)
            p_v[j, sl] = jnp.exp(r)
        pltpu.async_copy(p_v.at[j], den_s.at[dst_v.at[j]], ssem, add=True)
        pltpu.async_copy(ae_v.at[j], sae_s.at[dst_v.at[j]], ssem, add=True)
        pltpu.async_copy(ones_v, deg_s.at[dst_v.at[j]], ssem, add=True)
        return carry
    lax.fori_loop(0, _NCHUNK, _chunk, 0)

    pltpu.sync_copy(p_v, p_o.at[pl.ds(cb, _NCHUNK)])

    def _drain(j, carry):
        pltpu.make_async_copy(p_v.at[j], den_s.at[dst_v.at[j]], ssem).wait()
        pltpu.make_async_copy(ae_v.at[j], sae_s.at[dst_v.at[j]], ssem).wait()
        pltpu.make_async_copy(ones_v, deg_s.at[dst_v.at[j]], ssem).wait()
        return carry
    lax.fori_loop(0, _NCHUNK, _drain, 0)

    plsc.subcore_barrier()
    pltpu.sync_copy(den_s.at[pl.ds(base, _STRIPE)],
                    den_o.at[c, pl.ds(base, _STRIPE)])
    pltpu.sync_copy(sae_s.at[pl.ds(base, _STRIPE)],
                    sae_o.at[c, pl.ds(base, _STRIPE)])
    pltpu.sync_copy(deg_s.at[pl.ds(base, _STRIPE)],
                    deg_o.at[c, pl.ds(base, _STRIPE)])


_sc_scores = pl.kernel(
    _sc_scores_body,
    out_type=(
        jax.ShapeDtypeStruct((_NCHTOT, _CH), jnp.float32),
        jax.ShapeDtypeStruct((_NCORE, _NPAD), jnp.float32),
        jax.ShapeDtypeStruct((_NCORE, _NPAD), jnp.float32),
        jax.ShapeDtypeStruct((_NCORE, _NPAD), jnp.float32),
    ),
    mesh=plsc.VectorSubcoreMesh(core_axis_name="c", subcore_axis_name="s"),
    scratch_types=[
        pltpu.VMEM((_NCHUNK, _CH), jnp.int32),
        pltpu.VMEM((_NCHUNK, _CH), jnp.int32),
        pltpu.VMEM((_NCHUNK, _CH), jnp.float32),
        pltpu.VMEM((_NCHUNK, _CH), jnp.float32),
        pltpu.VMEM((_CH,), jnp.float32),
        pltpu.VMEM((_NPAD,), jnp.float32),
        pltpu.VMEM((_NPAD,), jnp.float32),
        pltpu.VMEM((_STRIPE,), jnp.float32),
        pltpu.VMEM_SHARED((_NPAD,), jnp.float32),
        pltpu.VMEM_SHARED((_NPAD,), jnp.float32),
        pltpu.VMEM_SHARED((_NPAD,), jnp.float32),
        pltpu.SemaphoreType.DMA,
    ],
    compiler_params=pltpu.CompilerParams(needs_layout_passes=False),
)


def _sc_msgs_body(src_hbm, dst_hbm, p_hbm, xp_hbm, acc_o,
                  src_v, dst_v, p_v, rows0_v, rows1_v, acc_s, gsem0, gsem1):
    c = lax.axis_index("c")
    s = lax.axis_index("s")
    base = s * _STRIPE
    rows = (rows0_v, rows1_v)
    gsem = (gsem0, gsem1)

    def _zrow(i, carry):
        for g in range(4):
            rows0_v[i, pl.ds(g * 32, 32)] = jnp.zeros((32,), jnp.bfloat16)
        return carry
    lax.fori_loop(0, _CH, _zrow, 0)
    for k in range(_STRIPE // _CH):
        pltpu.sync_copy(rows0_v, acc_s.at[pl.ds(base + k * _CH, _CH)])
    plsc.subcore_barrier()

    def _run(cb0, nsb):
        def _sblock(sb, carry):
            sb0 = cb0 + sb * _SB
            pltpu.sync_copy(src_hbm.at[pl.ds(sb0, _SB)], src_v)
            pltpu.sync_copy(dst_hbm.at[pl.ds(sb0, _SB)], dst_v)
            pltpu.sync_copy(p_hbm.at[pl.ds(sb0, _SB)], p_v)
            pltpu.async_copy(xp_hbm.at[src_v.at[0]], rows0_v, gsem0)

            def _pair(pr, carry2):
                for b in range(2):
                    j = pr * 2 + b
                    rb = rows[b]
                    pltpu.make_async_copy(xp_hbm.at[src_v.at[j]], rb,
                                          gsem[b]).wait()

                    @pl.when(j < _SB - 1)
                    def _():
                        pltpu.async_copy(xp_hbm.at[src_v.at[j + 1]],
                                         rows[1 - b], gsem[1 - b])

                    @plsc.parallel_loop(0, _CH, unroll=4)
                    def _rowscale(rr):
                        jj = jnp.full((16,), j, jnp.int32)
                        pb = plsc.load_gather(
                            p_v, [jj, jnp.full((16,), rr, jnp.int32)])
                        pbb = plsc.pack(pb, pb, format=plsc.PackFormat.INTERLEAVED)
                        for g in range(4):
                            sl = pl.ds(g * 32, 32)
                            rb[rr, sl] = rb[rr, sl] * pbb

                    pltpu.sync_copy(rb, acc_s.at[dst_v.at[j]], add=True)
                return carry2
            lax.fori_loop(0, _SB // 2, _pair, 0)
            return carry
        lax.fori_loop(0, nsb, _sblock, 0)

    @pl.when(c == 0)
    def _():
        _run(s * _MC0, _MC0 // _SB)

    @pl.when(c == 1)
    def _():
        _run(_NSUB * _MC0 + s * _MC1, _MC1 // _SB)

    plsc.subcore_barrier()
    pltpu.sync_copy(acc_s.at[pl.ds(base, _STRIPE)], acc_o.at[c, pl.ds(base, _STRIPE)])


_sc_msgs = pl.kernel(
    _sc_msgs_body,
    out_type=jax.ShapeDtypeStruct((_NCORE, _NPAD, _D), jnp.bfloat16),
    mesh=plsc.VectorSubcoreMesh(core_axis_name="c", subcore_axis_name="s"),
    scratch_types=[
        pltpu.VMEM((_SB, _CH), jnp.int32),
        pltpu.VMEM((_SB, _CH), jnp.int32),
        pltpu.VMEM((_SB, _CH), jnp.float32),
        pltpu.VMEM((_CH, _D), jnp.bfloat16),
        pltpu.VMEM((_CH, _D), jnp.bfloat16),
        pltpu.VMEM_SHARED((_NPAD, _D), jnp.bfloat16),
        pltpu.SemaphoreType.DMA,
        pltpu.SemaphoreType.DMA,
    ],
    compiler_params=pltpu.CompilerParams(needs_layout_passes=False,
                                         use_tc_tiling_on_sc=False),
)


def kernel(x, edge_index, edge_attr, batch, W1, att_src1, att_dst1, We1, att_e1,
           b1, W2, att_src2, att_dst2, We2, att_e2, b2, Wf, bf):
    ae1, ae2 = _edge_pro(edge_attr, We1, att_e1.reshape(_D, 1), We2,
                         att_e2.reshape(_D, 1))
    src, dst = _layout(
        edge_index[0].astype(jnp.int32).reshape(_NCHREAL, _CH),
        edge_index[1].astype(jnp.int32).reshape(_NCHREAL, _CH))

    def pad_n(v):
        return jnp.concatenate([v.reshape(_N), jnp.zeros((_NPAD - _N,), jnp.float32)])

    def v3(a):
        return a.reshape(_NCORE, _NPAD, 1)

    xp1, xp1b, asr1, adt1 = _node_pro(x, W1, att_src1.reshape(_D, 1),
                                      att_dst1.reshape(_D, 1))
    p1, den1, sae1, deg1 = _sc_scores(src, dst, ae1, pad_n(asr1), pad_n(adt1))
    acc1 = _sc_msgs(src, dst, p1, xp1b)

    xp2, xp2b, asr2, adt2 = _tc_mid(
        acc1, v3(den1), v3(sae1), v3(deg1), asr1, adt1, xp1,
        b1.reshape(1, _D), W2, att_src2.reshape(_D, 1), att_dst2.reshape(_D, 1))

    p2, den2, sae2, _ = _sc_scores(src, dst, ae2, pad_n(asr2), pad_n(adt2))
    acc2 = _sc_msgs(src, dst, p2, xp2b)

    return _tc_fin(
        acc2, v3(den2), v3(sae2), v3(deg1), asr2, adt2, xp2,
        b2.reshape(1, _D), batch.astype(jnp.int32).reshape(_N, 1), Wf,
        bf.reshape(1, _NCLS))


# speedup vs baseline: 1.1992x; 1.0459x over previous
import functools

import jax
import jax.numpy as jnp
from jax import lax
from jax.experimental import pallas as pl
from jax.experimental.pallas import tpu as pltpu
from jax.experimental.pallas import tpu_sc as plsc

_N = 10000
_E = 320000
_D = 128
_DE = 16
_NG = 64
_NCLS = 10

_NCORE = 2
_NSUB = 16
_CH = 128
_SB = 8
_MC0 = 120
_MC1 = 40
_NCHUNK = 80
_NCHTOT = _NCORE * _NSUB * _NCHUNK
_NCHREAL = _E // _CH
_NPAD = 10240
_STRIPE = _NPAD // _NSUB

_R = 1000
_GRID = _N // _R


def _node_pro_body(x_ref, w_ref, av_ref, ad_ref, xp_ref, xpb_ref, asr_ref, adt_ref):
    xp = jnp.dot(x_ref[...], w_ref[...], preferred_element_type=jnp.float32)
    xp_ref[...] = xp
    xpb_ref[...] = xp.astype(jnp.bfloat16)
    asr_ref[...] = jnp.dot(xp, av_ref[...], preferred_element_type=jnp.float32)
    adt_ref[...] = jnp.dot(xp, ad_ref[...], preferred_element_type=jnp.float32)


def _node_pro(x, W, av, ad):
    return pl.pallas_call(
        _node_pro_body,
        grid=(_GRID,),
        in_specs=[
            pl.BlockSpec((_R, _D), lambda i: (i, 0)),
            pl.BlockSpec((_D, _D), lambda i: (0, 0)),
            pl.BlockSpec((_D, 1), lambda i: (0, 0)),
            pl.BlockSpec((_D, 1), lambda i: (0, 0)),
        ],
        out_specs=[
            pl.BlockSpec((_R, _D), lambda i: (i, 0)),
            pl.BlockSpec((_R, _D), lambda i: (i, 0)),
            pl.BlockSpec((_R, 1), lambda i: (i, 0)),
            pl.BlockSpec((_R, 1), lambda i: (i, 0)),
        ],
        out_shape=[
            jax.ShapeDtypeStruct((_N, _D), jnp.float32),
            jax.ShapeDtypeStruct((_N, _D), jnp.bfloat16),
            jax.ShapeDtypeStruct((_N, 1), jnp.float32),
            jax.ShapeDtypeStruct((_N, 1), jnp.float32),
        ],
    )(x, W, av, ad)


_EB = 8192


def _edge_pro_body(ea_ref, we1_ref, ae1_ref, we2_ref, ae2_ref, o1_ref, o2_ref):
    i = pl.program_id(0)
    eidx = i * _EB + lax.broadcasted_iota(jnp.int32, (_EB, 1), 0)
    real = eidx < _E
    ea = ea_ref[...]
    for we_ref, atte_ref, o_ref in ((we1_ref, ae1_ref, o1_ref),
                                    (we2_ref, ae2_ref, o2_ref)):
        wv = jnp.dot(we_ref[...], atte_ref[...], preferred_element_type=jnp.float32)
        v = jnp.where(real, jnp.dot(ea, wv, preferred_element_type=jnp.float32),
                      -1e30)
        o_ref[...] = v.reshape(_EB // _CH, _CH)


def _edge_pro(edge_attr, We1, atte1, We2, atte2):
    return pl.pallas_call(
        _edge_pro_body,
        grid=(_NCHTOT * _CH // _EB,),
        in_specs=[
            pl.BlockSpec((_EB, _DE), lambda i: (i, 0)),
            pl.BlockSpec((_DE, _D), lambda i: (0, 0)),
            pl.BlockSpec((_D, 1), lambda i: (0, 0)),
            pl.BlockSpec((_DE, _D), lambda i: (0, 0)),
            pl.BlockSpec((_D, 1), lambda i: (0, 0)),
        ],
        out_specs=[pl.BlockSpec((_EB // _CH, _CH), lambda i: (i, 0))] * 2,
        out_shape=[jax.ShapeDtypeStruct((_NCHTOT, _CH), jnp.float32)] * 2,
    )(edge_attr, We1, atte1, We2, atte2)


def _layout_body(src_ref, dst_ref, srco_ref, dsto_ref):
    i = pl.program_id(0)
    rid = i * 512 + lax.broadcasted_iota(jnp.int32, (512, _CH), 0)
    real = rid < _NCHREAL
    srco_ref[...] = jnp.where(real, src_ref[...], 0)
    dsto_ref[...] = jnp.where(real, dst_ref[...], _N)


def _layout(src2d, dst2d):
    blk = pl.BlockSpec((512, _CH), lambda i: (i, 0))
    return pl.pallas_call(
        _layout_body,
        grid=(_NCHTOT // 512,),
        in_specs=[blk] * 2,
        out_specs=[blk] * 2,
        out_shape=[
            jax.ShapeDtypeStruct((_NCHTOT, _CH), jnp.int32),
            jax.ShapeDtypeStruct((_NCHTOT, _CH), jnp.int32),
        ],
    )(src2d, dst2d)


def _epilogue(acc0, acc1, den0, den1, sae0, sae1, dg0, dg1, asr, adt, xp, b_ref):
    def m2(r):
        return r[...].reshape(_R, _D).astype(jnp.float32)

    def v2(r):
        return r[...].reshape(_R, 1)

    deg = jnp.maximum(v2(dg0) + v2(dg1), 1.0)
    aloop = asr[...] + adt[...] + (v2(sae0) + v2(sae1)) / deg
    aloop = jnp.where(aloop >= 0.0, aloop, 0.2 * aloop)
    ploop = jnp.exp(aloop)
    invd = 1.0 / (v2(den0) + v2(den1) + ploop + 1e-16)
    h = (m2(acc0) + m2(acc1) + ploop * xp[...]) * invd + b_ref[...]
    return jnp.maximum(h, 0.0)


def _tc_mid_body(acc0, acc1, den0, den1, sae0, sae1, dg0, dg1, asr, adt, xp,
                 b_ref, w2_ref, av2_ref, ad2_ref, xp2_ref, xp2b_ref, as2_ref,
                 ad2o_ref):
    h = _epilogue(acc0, acc1, den0, den1, sae0, sae1, dg0, dg1, asr, adt, xp, b_ref)
    xp2 = jnp.dot(h, w2_ref[...], preferred_element_type=jnp.float32)
    xp2_ref[...] = xp2
    xp2b_ref[...] = xp2.astype(jnp.bfloat16)
    as2_ref[...] = jnp.dot(xp2, av2_ref[...], preferred_element_type=jnp.float32)
    ad2o_ref[...] = jnp.dot(xp2, ad2_ref[...], preferred_element_type=jnp.float32)


def _core_specs():
    acc0 = pl.BlockSpec((1, _R, _D), lambda i: (0, i, 0))
    acc1 = pl.BlockSpec((1, _R, _D), lambda i: (1, i, 0))
    v0 = pl.BlockSpec((1, _R, 1), lambda i: (0, i, 0))
    v1 = pl.BlockSpec((1, _R, 1), lambda i: (1, i, 0))
    return [acc0, acc1, v0, v1, v0, v1, v0, v1]


def _tc_mid(acc, den, sae, dg, asr, adt, xp, b, W2, av2, ad2):
    vec = pl.BlockSpec((_R, 1), lambda i: (i, 0))
    mat = pl.BlockSpec((_R, _D), lambda i: (i, 0))
    return pl.pallas_call(
        _tc_mid_body,
        grid=(_GRID,),
        in_specs=_core_specs() + [vec, vec] + [
            mat,
            pl.BlockSpec((1, _D), lambda i: (0, 0)),
            pl.BlockSpec((_D, _D), lambda i: (0, 0)),
            pl.BlockSpec((_D, 1), lambda i: (0, 0)),
            pl.BlockSpec((_D, 1), lambda i: (0, 0)),
        ],
        out_specs=[mat, mat, vec, vec],
        out_shape=[
            jax.ShapeDtypeStruct((_N, _D), jnp.float32),
            jax.ShapeDtypeStruct((_N, _D), jnp.bfloat16),
            jax.ShapeDtypeStruct((_N, 1), jnp.float32),
            jax.ShapeDtypeStruct((_N, 1), jnp.float32),
        ],
    )(acc, acc, den, den, sae, sae, dg, dg, asr, adt, xp, b, W2, av2, ad2)


def _tc_fin_body(acc0, acc1, den0, den1, sae0, sae1, dg0, dg1, asr, adt, xp,
                 b_ref, batch_ref, wf_ref, bf_ref, out_ref, pooled_s, cnt_s):
    i = pl.program_id(0)
    h = _epilogue(acc0, acc1, den0, den1, sae0, sae1, dg0, dg1, asr, adt, xp, b_ref)
    oh = (batch_ref[...] == lax.broadcasted_iota(jnp.int32, (_R, _NG), 1)
          ).astype(jnp.float32)

    @pl.when(i == 0)
    def _():
        pooled_s[...] = jnp.zeros_like(pooled_s)
        cnt_s[...] = jnp.zeros_like(cnt_s)

    dn = (((0,), (0,)), ((), ()))
    pooled_s[...] += lax.dot_general(oh, h, dn, preferred_element_type=jnp.float32)
    cnt_s[...] += lax.dot_general(oh, jnp.ones((_R, _D), jnp.float32), dn,
                                  preferred_element_type=jnp.float32)

    @pl.when(i == _GRID - 1)
    def _():
        pooled = pooled_s[...] / jnp.maximum(cnt_s[...], 1.0)
        logits = jnp.dot(pooled, wf_ref[...], preferred_element_type=jnp.float32) + bf_ref[...]
        m = jnp.max(logits, axis=1, keepdims=True)
        e = jnp.exp(logits - m)
        out_ref[...] = e / jnp.sum(e, axis=1, keepdims=True)


def _tc_fin(acc, den, sae, dg, asr, adt, xp, b, batch, Wf, bf):
    vec = pl.BlockSpec((_R, 1), lambda i: (i, 0))
    mat = pl.BlockSpec((_R, _D), lambda i: (i, 0))
    return pl.pallas_call(
        _tc_fin_body,
        grid=(_GRID,),
        in_specs=_core_specs() + [vec, vec] + [
            mat,
            pl.BlockSpec((1, _D), lambda i: (0, 0)),
            pl.BlockSpec((_R, 1), lambda i: (i, 0)),
            pl.BlockSpec((_D, _NCLS), lambda i: (0, 0)),
            pl.BlockSpec((1, _NCLS), lambda i: (0, 0)),
        ],
        out_specs=pl.BlockSpec((_NG, _NCLS), lambda i: (0, 0)),
        out_shape=jax.ShapeDtypeStruct((_NG, _NCLS), jnp.float32),
        scratch_shapes=[
            pltpu.VMEM((_NG, _D), jnp.float32),
            pltpu.VMEM((_NG, _D), jnp.float32),
        ],
    )(acc, acc, den, den, sae, sae, dg, dg, asr, adt, xp, b, batch, Wf, bf)


def _sc_scores_body(src_hbm, dst_hbm, ae_hbm, asrc_hbm, adst_hbm,
                    p_o, den_o, sae_o, deg_o,
                    src_v, dst_v, ae_v, p_v, ones_v, asrc_v, adst_v, zvec_v,
                    den_s, sae_s, deg_s, ssem):
    c = lax.axis_index("c")
    s = lax.axis_index("s")
    base = s * _STRIPE
    cb = (c * _NSUB + s) * _NCHUNK

    pltpu.sync_copy(src_hbm.at[pl.ds(cb, _NCHUNK)], src_v)
    pltpu.sync_copy(dst_hbm.at[pl.ds(cb, _NCHUNK)], dst_v)
    pltpu.sync_copy(ae_hbm.at[pl.ds(cb, _NCHUNK)], ae_v)
    pltpu.sync_copy(asrc_hbm, asrc_v)
    pltpu.sync_copy(adst_hbm, adst_v)

    def _zvec(i, carry):
        zvec_v[pl.ds(i * 16, 16)] = jnp.zeros((16,), jnp.float32)
        return carry
    lax.fori_loop(0, _STRIPE // 16, _zvec, 0)
    for g in range(8):
        ones_v[pl.ds(g * 16, 16)] = jnp.ones((16,), jnp.float32)

    pltpu.sync_copy(zvec_v, den_s.at[pl.ds(base, _STRIPE)])
    pltpu.sync_copy(zvec_v, sae_s.at[pl.ds(base, _STRIPE)])
    pltpu.sync_copy(zvec_v, deg_s.at[pl.ds(base, _STRIPE)])
    plsc.subcore_barrier()

    def _chunk(j, carry):
        for g in range(8):
            sl = pl.ds(g * 16, 16)
            si = src_v[j, sl]
            di = dst_v[j, sl]
            r = (plsc.load_gather(asrc_v, [si]) + plsc.load_gather(adst_v, [di])
                 + ae_v[j, sl])
            r = jnp.where(r >= 0.0, r, 0.2 * r)
            p_v[j, sl] = jnp.exp(r)
        pltpu.async_copy(p_v.at[j], den_s.at[dst_v.at[j]], ssem, add=True)
        pltpu.async_copy(ae_v.at[j], sae_s.at[dst_v.at[j]], ssem, add=True)
        pltpu.async_copy(ones_v, deg_s.at[dst_v.at[j]], ssem, add=True)
        return carry
    lax.fori_loop(0, _NCHUNK, _chunk, 0)

    pltpu.sync_copy(p_v, p_o.at[pl.ds(cb, _NCHUNK)])

    def _drain(j, carry):
        pltpu.make_async_copy(p_v.at[j], den_s.at[dst_v.at[j]], ssem).wait()
        pltpu.make_async_copy(ae_v.at[j], sae_s.at[dst_v.at[j]], ssem).wait()
        pltpu.make_async_copy(ones_v, deg_s.at[dst_v.at[j]], ssem).wait()
        return carry
    lax.fori_loop(0, _NCHUNK, _drain, 0)

    plsc.subcore_barrier()
    pltpu.sync_copy(den_s.at[pl.ds(base, _STRIPE)],
                    den_o.at[c, pl.ds(base, _STRIPE)])
    pltpu.sync_copy(sae_s.at[pl.ds(base, _STRIPE)],
                    sae_o.at[c, pl.ds(base, _STRIPE)])
    pltpu.sync_copy(deg_s.at[pl.ds(base, _STRIPE)],
                    deg_o.at[c, pl.ds(base, _STRIPE)])


_sc_scores = pl.kernel(
    _sc_scores_body,
    out_type=(
        jax.ShapeDtypeStruct((_NCHTOT, _CH), jnp.float32),
        jax.ShapeDtypeStruct((_NCORE, _NPAD), jnp.float32),
        jax.ShapeDtypeStruct((_NCORE, _NPAD), jnp.float32),
        jax.ShapeDtypeStruct((_NCORE, _NPAD), jnp.float32),
    ),
    mesh=plsc.VectorSubcoreMesh(core_axis_name="c", subcore_axis_name="s"),
    scratch_types=[
        pltpu.VMEM((_NCHUNK, _CH), jnp.int32),
        pltpu.VMEM((_NCHUNK, _CH), jnp.int32),
        pltpu.VMEM((_NCHUNK, _CH), jnp.float32),
        pltpu.VMEM((_NCHUNK, _CH), jnp.float32),
        pltpu.VMEM((_CH,), jnp.float32),
        pltpu.VMEM((_NPAD,), jnp.float32),
        pltpu.VMEM((_NPAD,), jnp.float32),
        pltpu.VMEM((_STRIPE,), jnp.float32),
        pltpu.VMEM_SHARED((_NPAD,), jnp.float32),
        pltpu.VMEM_SHARED((_NPAD,), jnp.float32),
        pltpu.VMEM_SHARED((_NPAD,), jnp.float32),
        pltpu.SemaphoreType.DMA,
    ],
    compiler_params=pltpu.CompilerParams(needs_layout_passes=False),
)


def _sc_msgs_body(src_hbm, dst_hbm, p_hbm, xp_hbm, acc_o,
                  src_v, dst_v, p_v, rows0_v, rows1_v, acc_s, gsem0, gsem1):
    c = lax.axis_index("c")
    s = lax.axis_index("s")
    base = s * _STRIPE
    rows = (rows0_v, rows1_v)
    gsem = (gsem0, gsem1)

    def _zrow(i, carry):
        for g in range(4):
            rows0_v[i, pl.ds(g * 32, 32)] = jnp.zeros((32,), jnp.bfloat16)
        return carry
    lax.fori_loop(0, _CH, _zrow, 0)
    for k in range(_STRIPE // _CH):
        pltpu.sync_copy(rows0_v, acc_s.at[pl.ds(base + k * _CH, _CH)])
    plsc.subcore_barrier()

    def _run(cb0, nsb):
        def _sblock(sb, carry):
            sb0 = cb0 + sb * _SB
            pltpu.sync_copy(src_hbm.at[pl.ds(sb0, _SB)], src_v)
            pltpu.sync_copy(dst_hbm.at[pl.ds(sb0, _SB)], dst_v)
            pltpu.sync_copy(p_hbm.at[pl.ds(sb0, _SB)], p_v)
            pltpu.async_copy(xp_hbm.at[src_v.at[0]], rows0_v, gsem0)

            def _pair(pr, carry2):
                for b in range(2):
                    j = pr * 2 + b
                    rb = rows[b]
                    pltpu.make_async_copy(xp_hbm.at[src_v.at[j]], rb,
                                          gsem[b]).wait()

                    @pl.when(j < _SB - 1)
                    def _():
                        pltpu.async_copy(xp_hbm.at[src_v.at[j + 1]],
                                         rows[1 - b], gsem[1 - b])

                    @plsc.parallel_loop(0, _CH, unroll=4)
                    def _rowscale(rr):
                        jj = jnp.full((16,), j, jnp.int32)
                        pb = plsc.load_gather(
                            p_v, [jj, jnp.full((16,), rr, jnp.int32)])
                        pbb = plsc.pack(pb, pb, format=plsc.PackFormat.INTERLEAVED)
                        for g in range(4):
                            sl = pl.ds(g * 32, 32)
                            rb[rr, sl] = rb[rr, sl] * pbb

                    pltpu.sync_copy(rb, acc_s.at[dst_v.at[j]], add=True)
                return carry2
            lax.fori_loop(0, _SB // 2, _pair, 0)
            return carry
        lax.fori_loop(0, nsb, _sblock, 0)

    @pl.when(c == 0)
    def _():
        _run(s * _MC0, _MC0 // _SB)

    @pl.when(c == 1)
    def _():
        _run(_NSUB * _MC0 + s * _MC1, _MC1 // _SB)

    plsc.subcore_barrier()
    pltpu.sync_copy(acc_s.at[pl.ds(base, _STRIPE)], acc_o.at[c, pl.ds(base, _STRIPE)])


_sc_msgs = pl.kernel(
    _sc_msgs_body,
    out_type=jax.ShapeDtypeStruct((_NCORE, _NPAD, _D), jnp.bfloat16),
    mesh=plsc.VectorSubcoreMesh(core_axis_name="c", subcore_axis_name="s"),
    scratch_types=[
        pltpu.VMEM((_SB, _CH), jnp.int32),
        pltpu.VMEM((_SB, _CH), jnp.int32),
        pltpu.VMEM((_SB, _CH), jnp.float32),
        pltpu.VMEM((_CH, _D), jnp.bfloat16),
        pltpu.VMEM((_CH, _D), jnp.bfloat16),
        pltpu.VMEM_SHARED((_NPAD, _D), jnp.bfloat16),
        pltpu.SemaphoreType.DMA,
        pltpu.SemaphoreType.DMA,
    ],
    compiler_params=pltpu.CompilerParams(needs_layout_passes=False,
                                         use_tc_tiling_on_sc=False),
)


def kernel(x, edge_index, edge_attr, batch, W1, att_src1, att_dst1, We1, att_e1,
           b1, W2, att_src2, att_dst2, We2, att_e2, b2, Wf, bf):
    ae1, ae2 = _edge_pro(edge_attr, We1, att_e1.reshape(_D, 1), We2,
                         att_e2.reshape(_D, 1))
    src, dst = _layout(
        edge_index[0].astype(jnp.int32).reshape(_NCHREAL, _CH),
        edge_index[1].astype(jnp.int32).reshape(_NCHREAL, _CH))

    def pad_n(v):
        return jnp.concatenate([v.reshape(_N), jnp.zeros((_NPAD - _N,), jnp.float32)])

    def v3(a):
        return a.reshape(_NCORE, _NPAD, 1)

    xp1, xp1b, asr1, adt1 = _node_pro(x, W1, att_src1.reshape(_D, 1),
                                      att_dst1.reshape(_D, 1))
    p1, den1, sae1, deg1 = _sc_scores(src, dst, ae1, pad_n(asr1), pad_n(adt1))
    acc1 = _sc_msgs(src, dst, p1, xp1b)

    xp2, xp2b, asr2, adt2 = _tc_mid(
        acc1, v3(den1), v3(sae1), v3(deg1), asr1, adt1, xp1,
        b1.reshape(1, _D), W2, att_src2.reshape(_D, 1), att_dst2.reshape(_D, 1))

    p2, den2, sae2, _ = _sc_scores(src, dst, ae2, pad_n(asr2), pad_n(adt2))
    acc2 = _sc_msgs(src, dst, p2, xp2b)

    return _tc_fin(
        acc2, v3(den2), v3(sae2), v3(deg1), asr2, adt2, xp2,
        b2.reshape(1, _D), batch.astype(jnp.int32).reshape(_N, 1), Wf,
        bf.reshape(1, _NCLS))


# speedup vs baseline: 1.2919x; 1.0773x over previous
import functools

import jax
import jax.numpy as jnp
from jax import lax
from jax.experimental import pallas as pl
from jax.experimental.pallas import tpu as pltpu
from jax.experimental.pallas import tpu_sc as plsc

_N = 10000
_E = 320000
_D = 128
_DE = 16
_NG = 64
_NCLS = 10

_NCORE = 2
_NSUB = 16
_CH = 128
_SB = 8
_MC0 = 120
_MC1 = 40
_NCHUNK = 80
_NCHTOT = _NCORE * _NSUB * _NCHUNK
_NCHREAL = _E // _CH
_NPAD = 10240
_STRIPE = _NPAD // _NSUB

_R = 1000
_GRID = _N // _R


def _node_pro_body(x_ref, w_ref, av_ref, ad_ref, xp_ref, xpb_ref, asr_ref, adt_ref):
    xp = jnp.dot(x_ref[...], w_ref[...], preferred_element_type=jnp.float32)
    xp_ref[...] = xp
    xpb_ref[...] = xp.astype(jnp.bfloat16)
    asr_ref[...] = jnp.dot(xp, av_ref[...], preferred_element_type=jnp.float32)
    adt_ref[...] = jnp.dot(xp, ad_ref[...], preferred_element_type=jnp.float32)


def _node_pro(x, W, av, ad):
    return pl.pallas_call(
        _node_pro_body,
        grid=(_GRID,),
        in_specs=[
            pl.BlockSpec((_R, _D), lambda i: (i, 0)),
            pl.BlockSpec((_D, _D), lambda i: (0, 0)),
            pl.BlockSpec((_D, 1), lambda i: (0, 0)),
            pl.BlockSpec((_D, 1), lambda i: (0, 0)),
        ],
        out_specs=[
            pl.BlockSpec((_R, _D), lambda i: (i, 0)),
            pl.BlockSpec((_R, _D), lambda i: (i, 0)),
            pl.BlockSpec((_R, 1), lambda i: (i, 0)),
            pl.BlockSpec((_R, 1), lambda i: (i, 0)),
        ],
        out_shape=[
            jax.ShapeDtypeStruct((_N, _D), jnp.float32),
            jax.ShapeDtypeStruct((_N, _D), jnp.bfloat16),
            jax.ShapeDtypeStruct((_N, 1), jnp.float32),
            jax.ShapeDtypeStruct((_N, 1), jnp.float32),
        ],
    )(x, W, av, ad)


def _edge_pro_body(ea_ref, we1_ref, ae1_ref, we2_ref, ae2_ref, o1_ref, o2_ref):
    cidx = lax.broadcasted_iota(jnp.int32, (_D, 8), 0)
    jidx = lax.broadcasted_iota(jnp.int32, (_D, 8), 1)
    m = (cidx // _DE == jidx).astype(jnp.float32)
    ea = ea_ref[...]
    for we_ref, atte_ref, o_ref in ((we1_ref, ae1_ref, o1_ref),
                                    (we2_ref, ae2_ref, o2_ref)):
        wv = jnp.dot(we_ref[...], atte_ref[...], preferred_element_type=jnp.float32)
        wt = jnp.concatenate([wv] * 8, axis=0)
        o_ref[...] = jnp.dot(ea, wt * m, preferred_element_type=jnp.float32)


def _edge_pro(ea_r, We1, atte1, We2, atte2):
    return pl.pallas_call(
        _edge_pro_body,
        grid=(5,),
        in_specs=[
            pl.BlockSpec((8000, _D), lambda i: (i, 0)),
            pl.BlockSpec((_DE, _D), lambda i: (0, 0)),
            pl.BlockSpec((_D, 1), lambda i: (0, 0)),
            pl.BlockSpec((_DE, _D), lambda i: (0, 0)),
            pl.BlockSpec((_D, 1), lambda i: (0, 0)),
        ],
        out_specs=[pl.BlockSpec((8000, 8), lambda i: (i, 0))] * 2,
        out_shape=[jax.ShapeDtypeStruct((_E // 8, 8), jnp.float32)] * 2,
    )(ea_r, We1, atte1, We2, atte2)


def _layout_body(src_ref, dst_ref, ae1_ref, ae2_ref,
                 srco_ref, dsto_ref, ae1o_ref, ae2o_ref):
    i = pl.program_id(0)
    rid = i * 512 + lax.broadcasted_iota(jnp.int32, (512, _CH), 0)
    real = rid < _NCHREAL
    srco_ref[...] = jnp.where(real, src_ref[...], 0)
    dsto_ref[...] = jnp.where(real, dst_ref[...], _N)
    ae1o_ref[...] = jnp.where(real, ae1_ref[...], -1e30)
    ae2o_ref[...] = jnp.where(real, ae2_ref[...], -1e30)


def _layout(src2d, dst2d, ae1_2d, ae2_2d):
    blk = pl.BlockSpec((512, _CH), lambda i: (i, 0))
    return pl.pallas_call(
        _layout_body,
        grid=(_NCHTOT // 512,),
        in_specs=[blk] * 4,
        out_specs=[blk] * 4,
        out_shape=[
            jax.ShapeDtypeStruct((_NCHTOT, _CH), jnp.int32),
            jax.ShapeDtypeStruct((_NCHTOT, _CH), jnp.int32),
            jax.ShapeDtypeStruct((_NCHTOT, _CH), jnp.float32),
            jax.ShapeDtypeStruct((_NCHTOT, _CH), jnp.float32),
        ],
    )(src2d, dst2d, ae1_2d, ae2_2d)


def _epilogue(acc0, acc1, den0, den1, sae0, sae1, dg0, dg1, asr, adt, xp, b_ref):
    def m2(r):
        return r[...].reshape(_R, _D).astype(jnp.float32)

    def v2(r):
        return r[...].reshape(_R, 1)

    deg = jnp.maximum(v2(dg0) + v2(dg1), 1.0)
    aloop = asr[...] + adt[...] + (v2(sae0) + v2(sae1)) / deg
    aloop = jnp.where(aloop >= 0.0, aloop, 0.2 * aloop)
    ploop = jnp.exp(aloop)
    invd = 1.0 / (v2(den0) + v2(den1) + ploop + 1e-16)
    h = (m2(acc0) + m2(acc1) + ploop * xp[...]) * invd + b_ref[...]
    return jnp.maximum(h, 0.0)


def _tc_mid_body(acc0, acc1, den0, den1, sae0, sae1, dg0, dg1, asr, adt, xp,
                 b_ref, w2_ref, av2_ref, ad2_ref, xp2_ref, xp2b_ref, as2_ref,
                 ad2o_ref):
    h = _epilogue(acc0, acc1, den0, den1, sae0, sae1, dg0, dg1, asr, adt, xp, b_ref)
    xp2 = jnp.dot(h, w2_ref[...], preferred_element_type=jnp.float32)
    xp2_ref[...] = xp2
    xp2b_ref[...] = xp2.astype(jnp.bfloat16)
    as2_ref[...] = jnp.dot(xp2, av2_ref[...], preferred_element_type=jnp.float32)
    ad2o_ref[...] = jnp.dot(xp2, ad2_ref[...], preferred_element_type=jnp.float32)


def _core_specs():
    acc0 = pl.BlockSpec((1, _R, _D), lambda i: (0, i, 0))
    acc1 = pl.BlockSpec((1, _R, _D), lambda i: (1, i, 0))
    v0 = pl.BlockSpec((1, _R, 1), lambda i: (0, i, 0))
    v1 = pl.BlockSpec((1, _R, 1), lambda i: (1, i, 0))
    return [acc0, acc1, v0, v1, v0, v1, v0, v1]


def _tc_mid(acc, den, sae, dg, asr, adt, xp, b, W2, av2, ad2):
    vec = pl.BlockSpec((_R, 1), lambda i: (i, 0))
    mat = pl.BlockSpec((_R, _D), lambda i: (i, 0))
    return pl.pallas_call(
        _tc_mid_body,
        grid=(_GRID,),
        in_specs=_core_specs() + [vec, vec] + [
            mat,
            pl.BlockSpec((1, _D), lambda i: (0, 0)),
            pl.BlockSpec((_D, _D), lambda i: (0, 0)),
            pl.BlockSpec((_D, 1), lambda i: (0, 0)),
            pl.BlockSpec((_D, 1), lambda i: (0, 0)),
        ],
        out_specs=[mat, mat, vec, vec],
        out_shape=[
            jax.ShapeDtypeStruct((_N, _D), jnp.float32),
            jax.ShapeDtypeStruct((_N, _D), jnp.bfloat16),
            jax.ShapeDtypeStruct((_N, 1), jnp.float32),
            jax.ShapeDtypeStruct((_N, 1), jnp.float32),
        ],
    )(acc, acc, den, den, sae, sae, dg, dg, asr, adt, xp, b, W2, av2, ad2)


def _tc_fin_body(acc0, acc1, den0, den1, sae0, sae1, dg0, dg1, asr, adt, xp,
                 b_ref, batch_ref, wf_ref, bf_ref, out_ref, pooled_s, cnt_s):
    i = pl.program_id(0)
    h = _epilogue(acc0, acc1, den0, den1, sae0, sae1, dg0, dg1, asr, adt, xp, b_ref)
    oh = (batch_ref[...] == lax.broadcasted_iota(jnp.int32, (_R, _NG), 1)
          ).astype(jnp.float32)

    @pl.when(i == 0)
    def _():
        pooled_s[...] = jnp.zeros_like(pooled_s)
        cnt_s[...] = jnp.zeros_like(cnt_s)

    dn = (((0,), (0,)), ((), ()))
    pooled_s[...] += lax.dot_general(oh, h, dn, preferred_element_type=jnp.float32)
    cnt_s[...] += lax.dot_general(oh, jnp.ones((_R, _D), jnp.float32), dn,
                                  preferred_element_type=jnp.float32)

    @pl.when(i == _GRID - 1)
    def _():
        pooled = pooled_s[...] / jnp.maximum(cnt_s[...], 1.0)
        logits = jnp.dot(pooled, wf_ref[...], preferred_element_type=jnp.float32) + bf_ref[...]
        m = jnp.max(logits, axis=1, keepdims=True)
        e = jnp.exp(logits - m)
        out_ref[...] = e / jnp.sum(e, axis=1, keepdims=True)


def _tc_fin(acc, den, sae, dg, asr, adt, xp, b, batch, Wf, bf):
    vec = pl.BlockSpec((_R, 1), lambda i: (i, 0))
    mat = pl.BlockSpec((_R, _D), lambda i: (i, 0))
    return pl.pallas_call(
        _tc_fin_body,
        grid=(_GRID,),
        in_specs=_core_specs() + [vec, vec] + [
            mat,
            pl.BlockSpec((1, _D), lambda i: (0, 0)),
            pl.BlockSpec((_R, 1), lambda i: (i, 0)),
            pl.BlockSpec((_D, _NCLS), lambda i: (0, 0)),
            pl.BlockSpec((1, _NCLS), lambda i: (0, 0)),
        ],
        out_specs=pl.BlockSpec((_NG, _NCLS), lambda i: (0, 0)),
        out_shape=jax.ShapeDtypeStruct((_NG, _NCLS), jnp.float32),
        scratch_shapes=[
            pltpu.VMEM((_NG, _D), jnp.float32),
            pltpu.VMEM((_NG, _D), jnp.float32),
        ],
    )(acc, acc, den, den, sae, sae, dg, dg, asr, adt, xp, b, batch, Wf, bf)


def _sc_scores_body(src_hbm, dst_hbm, ae_hbm, asrc_hbm, adst_hbm,
                    p_o, den_o, sae_o, deg_o,
                    src_v, dst_v, ae_v, p_v, ones_v, asrc_v, adst_v, zvec_v,
                    den_s, sae_s, deg_s, ssem):
    c = lax.axis_index("c")
    s = lax.axis_index("s")
    base = s * _STRIPE
    cb = (c * _NSUB + s) * _NCHUNK

    pltpu.sync_copy(src_hbm.at[pl.ds(cb, _NCHUNK)], src_v)
    pltpu.sync_copy(dst_hbm.at[pl.ds(cb, _NCHUNK)], dst_v)
    pltpu.sync_copy(ae_hbm.at[pl.ds(cb, _NCHUNK)], ae_v)
    pltpu.sync_copy(asrc_hbm, asrc_v)
    pltpu.sync_copy(adst_hbm, adst_v)

    def _zvec(i, carry):
        zvec_v[pl.ds(i * 16, 16)] = jnp.zeros((16,), jnp.float32)
        return carry
    lax.fori_loop(0, _STRIPE // 16, _zvec, 0)
    for g in range(8):
        ones_v[pl.ds(g * 16, 16)] = jnp.ones((16,), jnp.float32)

    pltpu.sync_copy(zvec_v, den_s.at[pl.ds(base, _STRIPE)])
    pltpu.sync_copy(zvec_v, sae_s.at[pl.ds(base, _STRIPE)])
    pltpu.sync_copy(zvec_v, deg_s.at[pl.ds(base, _STRIPE)])
    plsc.subcore_barrier()

    def _chunk(j, carry):
        for g in range(8):
            sl = pl.ds(g * 16, 16)
            si = src_v[j, sl]
            di = dst_v[j, sl]
            r = (plsc.load_gather(asrc_v, [si]) + plsc.load_gather(adst_v, [di])
                 + ae_v[j, sl])
            r = jnp.where(r >= 0.0, r, 0.2 * r)
            p_v[j, sl] = jnp.exp(r)
        pltpu.async_copy(p_v.at[j], den_s.at[dst_v.at[j]], ssem, add=True)
        pltpu.async_copy(ae_v.at[j], sae_s.at[dst_v.at[j]], ssem, add=True)
        pltpu.async_copy(ones_v, deg_s.at[dst_v.at[j]], ssem, add=True)
        return carry
    lax.fori_loop(0, _NCHUNK, _chunk, 0)

    pltpu.sync_copy(p_v, p_o.at[pl.ds(cb, _NCHUNK)])

    def _drain(j, carry):
        pltpu.make_async_copy(p_v.at[j], den_s.at[dst_v.at[j]], ssem).wait()
        pltpu.make_async_copy(ae_v.at[j], sae_s.at[dst_v.at[j]], ssem).wait()
        pltpu.make_async_copy(ones_v, deg_s.at[dst_v.at[j]], ssem).wait()
        return carry
    lax.fori_loop(0, _NCHUNK, _drain, 0)

    plsc.subcore_barrier()
    pltpu.sync_copy(den_s.at[pl.ds(base, _STRIPE)],
                    den_o.at[c, pl.ds(base, _STRIPE)])
    pltpu.sync_copy(sae_s.at[pl.ds(base, _STRIPE)],
                    sae_o.at[c, pl.ds(base, _STRIPE)])
    pltpu.sync_copy(deg_s.at[pl.ds(base, _STRIPE)],
                    deg_o.at[c, pl.ds(base, _STRIPE)])


_sc_scores = pl.kernel(
    _sc_scores_body,
    out_type=(
        jax.ShapeDtypeStruct((_NCHTOT, _CH), jnp.float32),
        jax.ShapeDtypeStruct((_NCORE, _NPAD), jnp.float32),
        jax.ShapeDtypeStruct((_NCORE, _NPAD), jnp.float32),
        jax.ShapeDtypeStruct((_NCORE, _NPAD), jnp.float32),
    ),
    mesh=plsc.VectorSubcoreMesh(core_axis_name="c", subcore_axis_name="s"),
    scratch_types=[
        pltpu.VMEM((_NCHUNK, _CH), jnp.int32),
        pltpu.VMEM((_NCHUNK, _CH), jnp.int32),
        pltpu.VMEM((_NCHUNK, _CH), jnp.float32),
        pltpu.VMEM((_NCHUNK, _CH), jnp.float32),
        pltpu.VMEM((_CH,), jnp.float32),
        pltpu.VMEM((_NPAD,), jnp.float32),
        pltpu.VMEM((_NPAD,), jnp.float32),
        pltpu.VMEM((_STRIPE,), jnp.float32),
        pltpu.VMEM_SHARED((_NPAD,), jnp.float32),
        pltpu.VMEM_SHARED((_NPAD,), jnp.float32),
        pltpu.VMEM_SHARED((_NPAD,), jnp.float32),
        pltpu.SemaphoreType.DMA,
    ],
    compiler_params=pltpu.CompilerParams(needs_layout_passes=False),
)


def _sc_msgs_body(src_hbm, dst_hbm, p_hbm, xp_hbm, acc_o,
                  src_v, dst_v, p_v, rows0_v, rows1_v, acc_s, gsem0, gsem1):
    c = lax.axis_index("c")
    s = lax.axis_index("s")
    base = s * _STRIPE
    rows = (rows0_v, rows1_v)
    gsem = (gsem0, gsem1)

    def _zrow(i, carry):
        for g in range(4):
            rows0_v[i, pl.ds(g * 32, 32)] = jnp.zeros((32,), jnp.bfloat16)
        return carry
    lax.fori_loop(0, _CH, _zrow, 0)
    for k in range(_STRIPE // _CH):
        pltpu.sync_copy(rows0_v, acc_s.at[pl.ds(base + k * _CH, _CH)])
    plsc.subcore_barrier()

    def _run(cb0, nsb):
        def _sblock(sb, carry):
            sb0 = cb0 + sb * _SB
            pltpu.sync_copy(src_hbm.at[pl.ds(sb0, _SB)], src_v)
            pltpu.sync_copy(dst_hbm.at[pl.ds(sb0, _SB)], dst_v)
            pltpu.sync_copy(p_hbm.at[pl.ds(sb0, _SB)], p_v)
            pltpu.async_copy(xp_hbm.at[src_v.at[0]], rows0_v, gsem0)

            def _pair(pr, carry2):
                for b in range(2):
                    j = pr * 2 + b
                    rb = rows[b]
                    pltpu.make_async_copy(xp_hbm.at[src_v.at[j]], rb,
                                          gsem[b]).wait()

                    @pl.when(j < _SB - 1)
                    def _():
                        pltpu.async_copy(xp_hbm.at[src_v.at[j + 1]],
                                         rows[1 - b], gsem[1 - b])

                    @plsc.parallel_loop(0, _CH, unroll=4)
                    def _rowscale(rr):
                        jj = jnp.full((16,), j, jnp.int32)
                        pb = plsc.load_gather(
                            p_v, [jj, jnp.full((16,), rr, jnp.int32)])
                        pbb = plsc.pack(pb, pb, format=plsc.PackFormat.INTERLEAVED)
                        for g in range(4):
                            sl = pl.ds(g * 32, 32)
                            rb[rr, sl] = rb[rr, sl] * pbb

                    pltpu.sync_copy(rb, acc_s.at[dst_v.at[j]], add=True)
                return carry2
            lax.fori_loop(0, _SB // 2, _pair, 0)
            return carry
        lax.fori_loop(0, nsb, _sblock, 0)

    @pl.when(c == 0)
    def _():
        _run(s * _MC0, _MC0 // _SB)

    @pl.when(c == 1)
    def _():
        _run(_NSUB * _MC0 + s * _MC1, _MC1 // _SB)

    plsc.subcore_barrier()
    pltpu.sync_copy(acc_s.at[pl.ds(base, _STRIPE)], acc_o.at[c, pl.ds(base, _STRIPE)])


_sc_msgs = pl.kernel(
    _sc_msgs_body,
    out_type=jax.ShapeDtypeStruct((_NCORE, _NPAD, _D), jnp.bfloat16),
    mesh=plsc.VectorSubcoreMesh(core_axis_name="c", subcore_axis_name="s"),
    scratch_types=[
        pltpu.VMEM((_SB, _CH), jnp.int32),
        pltpu.VMEM((_SB, _CH), jnp.int32),
        pltpu.VMEM((_SB, _CH), jnp.float32),
        pltpu.VMEM((_CH, _D), jnp.bfloat16),
        pltpu.VMEM((_CH, _D), jnp.bfloat16),
        pltpu.VMEM_SHARED((_NPAD, _D), jnp.bfloat16),
        pltpu.SemaphoreType.DMA,
        pltpu.SemaphoreType.DMA,
    ],
    compiler_params=pltpu.CompilerParams(needs_layout_passes=False,
                                         use_tc_tiling_on_sc=False),
)


def kernel(x, edge_index, edge_attr, batch, W1, att_src1, att_dst1, We1, att_e1,
           b1, W2, att_src2, att_dst2, We2, att_e2, b2, Wf, bf):
    ae1_2d, ae2_2d = _edge_pro(edge_attr.reshape(_E // 8, _D), We1,
                               att_e1.reshape(_D, 1), We2, att_e2.reshape(_D, 1))
    src, dst, ae1, ae2 = _layout(
        edge_index[0].astype(jnp.int32).reshape(_NCHREAL, _CH),
        edge_index[1].astype(jnp.int32).reshape(_NCHREAL, _CH),
        ae1_2d.reshape(_NCHREAL, _CH), ae2_2d.reshape(_NCHREAL, _CH))

    def pad_n(v):
        return jnp.concatenate([v.reshape(_N), jnp.zeros((_NPAD - _N,), jnp.float32)])

    def v3(a):
        return a.reshape(_NCORE, _NPAD, 1)

    xp1, xp1b, asr1, adt1 = _node_pro(x, W1, att_src1.reshape(_D, 1),
                                      att_dst1.reshape(_D, 1))
    p1, den1, sae1, deg1 = _sc_scores(src, dst, ae1, pad_n(asr1), pad_n(adt1))
    acc1 = _sc_msgs(src, dst, p1, xp1b)

    xp2, xp2b, asr2, adt2 = _tc_mid(
        acc1, v3(den1), v3(sae1), v3(deg1), asr1, adt1, xp1,
        b1.reshape(1, _D), W2, att_src2.reshape(_D, 1), att_dst2.reshape(_D, 1))

    p2, den2, sae2, _ = _sc_scores(src, dst, ae2, pad_n(asr2), pad_n(adt2))
    acc2 = _sc_msgs(src, dst, p2, xp2b)

    return _tc_fin(
        acc2, v3(den2), v3(sae2), v3(deg1), asr2, adt2, xp2,
        b2.reshape(1, _D), batch.astype(jnp.int32).reshape(_N, 1), Wf,
        bf.reshape(1, _NCLS))
